# baseline (speedup 1.0000x reference)
import functools

import jax
import jax.numpy as jnp
from jax import lax
from jax.experimental import pallas as pl
from jax.experimental.pallas import tpu as pltpu
from jax.experimental.pallas import tpu_sc as plsc

BATCH = 16384
EMBED_DIM = 32
L = 16
NC = 2
NS = 16
NW = NC * NS
BPW = BATCH // NW
CHUNK = 128
NCHUNK = BPW // CHUNK


def _mesh():
    return plsc.VectorSubcoreMesh(core_axis_name="c", subcore_axis_name="s")


@functools.partial(
    pl.kernel,
    out_type=jax.ShapeDtypeStruct((BATCH,), jnp.float32),
    mesh=_mesh(),
    scratch_types=[
        pltpu.VMEM((BPW,), jnp.int32),
        pltpu.VMEM((BPW,), jnp.int32),
        pltpu.VMEM((BPW, EMBED_DIM), jnp.float32),
        pltpu.VMEM((BPW, EMBED_DIM), jnp.float32),
        pltpu.VMEM((4 * L,), jnp.float32),
        pltpu.VMEM((L,), jnp.float32),
        pltpu.VMEM((BPW,), jnp.float32),
        pltpu.SemaphoreType.DMA,
        pltpu.SemaphoreType.DMA,
    ],
    compiler_params=pltpu.CompilerParams(
        needs_layout_passes=False, use_tc_tiling_on_sc=False),
)
def _sc_kernel(users_hbm, movies_hbm, ut_hbm, mt_hbm, w_hbm, b16_hbm, out_hbm,
               uidx_v, midx_v, urows_v, mrows_v, w_v, b16_v, acc_v,
               usem, msem):
    wid = lax.axis_index("s") * NC + lax.axis_index("c")
    base = wid * BPW

    pltpu.sync_copy(users_hbm.at[pl.ds(base, BPW)], uidx_v)
    pltpu.sync_copy(movies_hbm.at[pl.ds(base, BPW)], midx_v)
    pltpu.sync_copy(w_hbm, w_v)
    pltpu.sync_copy(b16_hbm, b16_v)

    ucopies = []
    mcopies = []
    for j in range(NCHUNK):
        sl = pl.ds(j * CHUNK, CHUNK)
        ucopies.append(
            pltpu.async_copy(ut_hbm.at[uidx_v.at[sl]], urows_v.at[sl], usem))
        mcopies.append(
            pltpu.async_copy(mt_hbm.at[midx_v.at[sl]], mrows_v.at[sl], msem))

    w_u0 = w_v[pl.ds(0, L)]
    w_u1 = w_v[pl.ds(L, L)]
    w_m0 = w_v[pl.ds(2 * L, L)]
    w_m1 = w_v[pl.ds(3 * L, L)]
    b16 = b16_v[...]
    msk15 = lax.iota(jnp.int32, L) == (L - 1)

    def chunk_body(j, urows, mrows):
        ucopies[j].wait()
        mcopies[j].wait()

        def body(i, _):
            u0 = urows_v[i, pl.ds(0, L)]
            u1 = urows_v[i, pl.ds(L, L)]
            m0 = mrows_v[i, pl.ds(0, L)]
            m1 = mrows_v[i, pl.ds(L, L)]
            acc = u0 * w_u0 + u1 * w_u1 + m0 * w_m0 + m1 * w_m1 + b16
            tot = plsc.cumsum(acc)
            idxv = jnp.full((L,), i, dtype=jnp.int32)
            plsc.store_scatter(acc_v, [idxv], tot, mask=msk15)
            return 0

        lax.fori_loop(j * CHUNK, (j + 1) * CHUNK, body, 0, unroll=4)

    for j in range(NCHUNK):
        chunk_body(j, urows_v, mrows_v)

    pltpu.sync_copy(acc_v, out_hbm.at[pl.ds(base, BPW)])


def kernel(users, movies, user_table, movie_table, W, b):
    w_flat = W.reshape(4 * L).astype(jnp.float32)
    b16 = jnp.full((L,), b[0] / L, dtype=jnp.float32)
    out = _sc_kernel(users.astype(jnp.int32), movies.astype(jnp.int32),
                     user_table, movie_table, w_flat, b16)
    return out.reshape(BATCH, 1)
